# Initial kernel scaffold; baseline (speedup 1.0000x reference)
import jax
import jax.numpy as jnp
from jax.experimental import pallas as pl


def kernel(x, edge_index, edge_weight, W0, W1, Wm, bm):
    raise NotImplementedError("write your pallas kernel here")


# speedup vs baseline: 3.9079x; 3.9079x over previous
import functools

import jax
import jax.numpy as jnp
from jax import lax
from jax.experimental import pallas as pl
from jax.experimental.pallas import tpu as pltpu
from jax.experimental.pallas import tpu_sc as plsc

N_NODES = 10000
N_EDGES = 320000
D = 128
D_OUT = 64

NC = 2
NS = 16
NW = NC * NS
EPW = N_EDGES // NW
CHUNK = 80
NCHUNK = EPW // CHUNK
RPT = 640
ZROWS = 80


def _spmm_body(x_hbm, src_hbm, dst_hbm, ew_hbm, out_hbm,
               src_v, dst_v, ew_v, rows_v, zbuf_v, acc, sem):
    cid = lax.axis_index("c")
    sid = lax.axis_index("s")
    wid = cid * NS + sid

    def zrow(r, carry):
        for c in range(8):
            zbuf_v[r, pl.ds(c * 16, 16)] = jnp.zeros((16,), jnp.float32)
        return carry
    lax.fori_loop(0, ZROWS, zrow, 0)
    row0 = sid * RPT

    @pl.when(sid < NS - 1)
    def _zero_full():
        for j in range(RPT // ZROWS):
            pltpu.sync_copy(zbuf_v, acc.at[pl.ds(row0 + j * ZROWS, ZROWS)])

    @pl.when(sid == NS - 1)
    def _zero_last():
        for j in range(5):
            pltpu.sync_copy(zbuf_v, acc.at[pl.ds(row0 + j * ZROWS, ZROWS)])

    plsc.subcore_barrier()

    inv_n = jnp.float32(1.0 / N_NODES)

    def chunk_body(ci, carry):
        base = wid * EPW + ci * CHUNK
        pltpu.sync_copy(src_hbm.at[pl.ds(base, CHUNK)], src_v)
        pltpu.sync_copy(dst_hbm.at[pl.ds(base, CHUNK)], dst_v)
        pltpu.sync_copy(ew_hbm.at[pl.ds(base, CHUNK)], ew_v)
        pltpu.async_copy(x_hbm.at[src_v], rows_v, sem).wait()

        def group_body(g, inner):
            ew16 = ew_v[pl.ds(g * 16, 16)] * inv_n
            for j in range(16):
                w = ew16[j]
                e = g * 16 + j
                for c in range(8):
                    sl = pl.ds(c * 16, 16)
                    rows_v[e, sl] = rows_v[e, sl] * w
            return inner
        lax.fori_loop(0, CHUNK // 16, group_body, 0)

        pltpu.sync_copy(rows_v, acc.at[dst_v], add=True)
        return carry
    lax.fori_loop(0, NCHUNK, chunk_body, 0)

    plsc.subcore_barrier()

    @pl.when(sid < NS - 1)
    def _out_full():
        pltpu.sync_copy(acc.at[pl.ds(row0, RPT)],
                        out_hbm.at[pl.ds(cid * N_NODES + row0, RPT)])

    @pl.when(sid == NS - 1)
    def _out_last():
        pltpu.sync_copy(acc.at[pl.ds(row0, 400)],
                        out_hbm.at[pl.ds(cid * N_NODES + row0, 400)])


def _spmm(x, src, dst, ew):
    mesh = plsc.VectorSubcoreMesh(core_axis_name="c", subcore_axis_name="s")
    f = pl.kernel(
        _spmm_body,
        out_type=jax.ShapeDtypeStruct((NC * N_NODES, D), jnp.float32),
        mesh=mesh,
        scratch_types=[
            pltpu.VMEM((CHUNK,), jnp.int32),
            pltpu.VMEM((CHUNK,), jnp.int32),
            pltpu.VMEM((CHUNK,), jnp.float32),
            pltpu.VMEM((CHUNK, D), jnp.float32),
            pltpu.VMEM((ZROWS, D), jnp.float32),
            pltpu.VMEM_SHARED((N_NODES, D), jnp.float32),
            pltpu.SemaphoreType.DMA,
        ],
    )
    return f(x, src, dst, ew)


BR = 1000


def _combine_mm_body(pa_ref, pb_ref, u_ref, w0_ref, w1_ref, z_ref, acc_ref):
    z = pa_ref[...] + pb_ref[...]
    z_ref[...] = z
    acc_ref[...] = (
        jnp.dot(u_ref[...], w0_ref[...], preferred_element_type=jnp.float32)
        + jnp.dot(z, w1_ref[...], preferred_element_type=jnp.float32))


def _tc_combine_mm(pa, pb, u, w0, w1):
    grid = (N_NODES // BR,)
    blk = pl.BlockSpec((BR, D), lambda i: (i, 0))
    wblk = pl.BlockSpec((D, D), lambda i: (0, 0))
    return pl.pallas_call(
        _combine_mm_body,
        grid=grid,
        in_specs=[blk, blk, blk, wblk, wblk],
        out_specs=[blk, blk],
        out_shape=[jax.ShapeDtypeStruct((N_NODES, D), jnp.float32),
                   jax.ShapeDtypeStruct((N_NODES, D), jnp.float32)],
    )(pa, pb, u, w0, w1)


def _finish_body(acc_ref, pa_ref, pb_ref, w2_ref, y_ref):
    z2 = pa_ref[...] + pb_ref[...]
    y_ref[...] = jnp.maximum(
        acc_ref[...]
        + jnp.dot(z2, w2_ref[...], preferred_element_type=jnp.float32), 0.0)


def _tc_finish(acc, pa, pb, w2):
    grid = (N_NODES // BR,)
    blk = pl.BlockSpec((BR, D), lambda i: (i, 0))
    wblk = pl.BlockSpec((D, D), lambda i: (0, 0))
    return pl.pallas_call(
        _finish_body,
        grid=grid,
        in_specs=[blk, blk, blk, wblk],
        out_specs=blk,
        out_shape=jax.ShapeDtypeStruct((N_NODES, D), jnp.float32),
    )(acc, pa, pb, w2)


def _final_body(acc_ref, pa_ref, pb_ref, w2_ref, wm_ref, bm_ref, out_ref):
    z2 = pa_ref[...] + pb_ref[...]
    y2 = jnp.maximum(
        acc_ref[...]
        + jnp.dot(z2, w2_ref[...], preferred_element_type=jnp.float32), 0.0)
    out_ref[...] = (
        jnp.dot(y2, wm_ref[...], preferred_element_type=jnp.float32)
        + bm_ref[...])


def _tc_final(acc, pa, pb, w2, wm, bm):
    grid = (N_NODES // BR,)
    blk = pl.BlockSpec((BR, D), lambda i: (i, 0))
    wblk = pl.BlockSpec((D, D), lambda i: (0, 0))
    return pl.pallas_call(
        _final_body,
        grid=grid,
        in_specs=[blk, blk, blk, wblk,
                  pl.BlockSpec((D, D_OUT), lambda i: (0, 0)),
                  pl.BlockSpec((1, D_OUT), lambda i: (0, 0))],
        out_specs=pl.BlockSpec((BR, D_OUT), lambda i: (i, 0)),
        out_shape=jax.ShapeDtypeStruct((N_NODES, D_OUT), jnp.float32),
    )(acc, pa, pb, w2, wm, bm)


def kernel(x, edge_index, edge_weight, W0, W1, Wm, bm):
    src = edge_index[1].astype(jnp.int32)
    dst = edge_index[0].astype(jnp.int32)
    ew = edge_weight

    p1 = _spmm(x, src, dst, ew)
    z1, acc1 = _tc_combine_mm(p1[:N_NODES], p1[N_NODES:], x, W0[0], W0[1])
    p2 = _spmm(z1, src, dst, ew)
    y1 = _tc_finish(acc1, p2[:N_NODES], p2[N_NODES:], W0[2])

    p3 = _spmm(y1, src, dst, ew)
    h1, acc2 = _tc_combine_mm(p3[:N_NODES], p3[N_NODES:], y1, W1[0], W1[1])
    p4 = _spmm(h1, src, dst, ew)
    return _tc_final(acc2, p4[:N_NODES], p4[N_NODES:], W1[2], Wm,
                     bm.reshape(1, D_OUT))


# speedup vs baseline: 4.0024x; 1.0242x over previous
import functools

import jax
import jax.numpy as jnp
from jax import lax
from jax.experimental import pallas as pl
from jax.experimental.pallas import tpu as pltpu
from jax.experimental.pallas import tpu_sc as plsc

N_NODES = 10000
N_EDGES = 320000
D = 128
DH = 64
D_OUT = 64

NC = 2
NS = 16
EPT = N_EDGES // NS
CHUNK = 80
NCHUNK = EPT // CHUNK
NBUF = 5
GAHEAD = 3
RPT = 640
ZROWS = 80


def _spmm_body(xa_hbm, xb_hbm, src_hbm, dst_hbm, ew_hbm, outa_hbm, outb_hbm,
               src_b, dst_b, ew_b, rows_v, zbuf_v, acc, isem, gsem):
    cid = lax.axis_index("c")
    sid = lax.axis_index("s")

    def zrow(r, carry):
        for c in range(DH // 16):
            zbuf_v[r, pl.ds(c * 16, 16)] = jnp.zeros((16,), jnp.float32)
        return carry
    lax.fori_loop(0, ZROWS, zrow, 0)
    row0 = sid * RPT

    @pl.when(sid < NS - 1)
    def _zero_full():
        for j in range(RPT // ZROWS):
            pltpu.sync_copy(zbuf_v, acc.at[pl.ds(row0 + j * ZROWS, ZROWS)])

    @pl.when(sid == NS - 1)
    def _zero_last():
        for j in range(5):
            pltpu.sync_copy(zbuf_v, acc.at[pl.ds(row0 + j * ZROWS, ZROWS)])

    plsc.subcore_barrier()

    inv_n = jnp.float32(1.0 / N_NODES)

    def _issue_idx(ci, b):
        pltpu.async_copy(src_hbm.at[sid, ci], src_b[b], isem[b])
        pltpu.async_copy(dst_hbm.at[sid, ci], dst_b[b], isem[b])
        pltpu.async_copy(ew_hbm.at[sid, ci], ew_b[b], isem[b])

    def _wait_idx(ci, b):
        pltpu.make_async_copy(src_hbm.at[sid, ci], src_b[b], isem[b]).wait()
        pltpu.make_async_copy(dst_hbm.at[sid, ci], dst_b[b], isem[b]).wait()
        pltpu.make_async_copy(ew_hbm.at[sid, ci], ew_b[b], isem[b]).wait()

    def _issue_gather(b):
        @pl.when(cid == 0)
        def _ga():
            pltpu.async_copy(xa_hbm.at[src_b[b]], rows_v[b], gsem[b])

        @pl.when(cid == 1)
        def _gb():
            pltpu.async_copy(xb_hbm.at[src_b[b]], rows_v[b], gsem[b])

    def _wait_gather(b):
        @pl.when(cid == 0)
        def _wa():
            pltpu.make_async_copy(xa_hbm.at[src_b[b]], rows_v[b],
                                  gsem[b]).wait()

        @pl.when(cid == 1)
        def _wb():
            pltpu.make_async_copy(xb_hbm.at[src_b[b]], rows_v[b],
                                  gsem[b]).wait()

    for j in range(NBUF):
        _issue_idx(j, j)
    for j in range(GAHEAD):
        _wait_idx(j, j)
        _issue_gather(j)

    def ring_body(cg, carry):
        for b in range(NBUF):
            ci = cg * NBUF + b
            _wait_gather(b)

            bg = (b + GAHEAD) % NBUF

            @pl.when(ci + GAHEAD < NCHUNK)
            def _launch_ahead():
                _wait_idx(ci + GAHEAD, bg)
                _issue_gather(bg)

            def group_body(g, inner):
                ew16 = ew_b[b][pl.ds(g * 16, 16)] * inv_n
                for j in range(16):
                    w = ew16[j]
                    e = g * 16 + j
                    for c in range(DH // 16):
                        sl = pl.ds(c * 16, 16)
                        rows_v[b][e, sl] = rows_v[b][e, sl] * w
                return inner
            lax.fori_loop(0, CHUNK // 16, group_body, 0)

            pltpu.sync_copy(rows_v[b], acc.at[dst_b[b]], add=True)

            @pl.when(ci + NBUF < NCHUNK)
            def _prefetch_idx():
                _issue_idx(ci + NBUF, b)
        return carry
    lax.fori_loop(0, NCHUNK // NBUF, ring_body, 0)

    plsc.subcore_barrier()

    @pl.when(cid == 0)
    def _outa():
        @pl.when(sid < NS - 1)
        def _full():
            pltpu.sync_copy(acc.at[pl.ds(row0, RPT)],
                            outa_hbm.at[pl.ds(row0, RPT)])

        @pl.when(sid == NS - 1)
        def _last():
            pltpu.sync_copy(acc.at[pl.ds(row0, 400)],
                            outa_hbm.at[pl.ds(row0, 400)])

    @pl.when(cid == 1)
    def _outb():
        @pl.when(sid < NS - 1)
        def _full():
            pltpu.sync_copy(acc.at[pl.ds(row0, RPT)],
                            outb_hbm.at[pl.ds(row0, RPT)])

        @pl.when(sid == NS - 1)
        def _last():
            pltpu.sync_copy(acc.at[pl.ds(row0, 400)],
                            outb_hbm.at[pl.ds(row0, 400)])


def _spmm(xa, xb, src3, dst3, ew3):
    mesh = plsc.VectorSubcoreMesh(core_axis_name="c", subcore_axis_name="s")
    f = pl.kernel(
        _spmm_body,
        compiler_params=pltpu.CompilerParams(use_tc_tiling_on_sc=False),
        out_type=(jax.ShapeDtypeStruct((N_NODES, DH), jnp.float32),
                  jax.ShapeDtypeStruct((N_NODES, DH), jnp.float32)),
        mesh=mesh,
        scratch_types=[
            [pltpu.VMEM((CHUNK,), jnp.int32)] * NBUF,
            [pltpu.VMEM((CHUNK,), jnp.int32)] * NBUF,
            [pltpu.VMEM((CHUNK,), jnp.float32)] * NBUF,
            [pltpu.VMEM((CHUNK, DH), jnp.float32)] * NBUF,
            pltpu.VMEM((ZROWS, DH), jnp.float32),
            pltpu.VMEM_SHARED((N_NODES, DH), jnp.float32),
            [pltpu.SemaphoreType.DMA] * NBUF,
            [pltpu.SemaphoreType.DMA] * NBUF,
        ],
    )
    return f(xa, xb, src3, dst3, ew3)


BR = 1000


def _layer1_body(x_ref, za1_ref, zb1_ref, za2_ref, zb2_ref,
                 w00_ref, w01a_ref, w01b_ref, w02a_ref, w02b_ref,
                 ya_ref, yb_ref):
    def dot(a, b):
        return jnp.dot(a, b, preferred_element_type=jnp.float32)
    y = dot(x_ref[...], w00_ref[...])
    y += dot(za1_ref[...], w01a_ref[...]) + dot(zb1_ref[...], w01b_ref[...])
    y += dot(za2_ref[...], w02a_ref[...]) + dot(zb2_ref[...], w02b_ref[...])
    y = jnp.maximum(y, 0.0)
    ya_ref[...] = y[:, :DH]
    yb_ref[...] = y[:, DH:]


def _tc_layer1(x, za1, zb1, za2, zb2, W0):
    grid = (N_NODES // BR,)
    fblk = pl.BlockSpec((BR, D), lambda i: (i, 0))
    hblk = pl.BlockSpec((BR, DH), lambda i: (i, 0))
    wf = pl.BlockSpec((D, D), lambda i: (0, 0))
    wh = pl.BlockSpec((DH, D), lambda i: (0, 0))
    return pl.pallas_call(
        _layer1_body,
        grid=grid,
        in_specs=[fblk, hblk, hblk, hblk, hblk, wf, wh, wh, wh, wh],
        out_specs=[hblk, hblk],
        out_shape=[jax.ShapeDtypeStruct((N_NODES, DH), jnp.float32),
                   jax.ShapeDtypeStruct((N_NODES, DH), jnp.float32)],
    )(x, za1, zb1, za2, zb2,
      W0[0], W0[1, :DH], W0[1, DH:], W0[2, :DH], W0[2, DH:])


def _layer2_body(ya_ref, yb_ref, ha1_ref, hb1_ref, ha2_ref, hb2_ref,
                 w10_ref, w11a_ref, w11b_ref, w12a_ref, w12b_ref,
                 wm_ref, bm_ref, out_ref):
    def dot(a, b):
        return jnp.dot(a, b, preferred_element_type=jnp.float32)
    y1 = jnp.concatenate([ya_ref[...], yb_ref[...]], axis=1)
    y2 = dot(y1, w10_ref[...])
    y2 += dot(ha1_ref[...], w11a_ref[...]) + dot(hb1_ref[...], w11b_ref[...])
    y2 += dot(ha2_ref[...], w12a_ref[...]) + dot(hb2_ref[...], w12b_ref[...])
    y2 = jnp.maximum(y2, 0.0)
    out_ref[...] = dot(y2, wm_ref[...]) + bm_ref[...]


def _tc_layer2(ya, yb, ha1, hb1, ha2, hb2, W1, Wm, bm):
    grid = (N_NODES // BR,)
    hblk = pl.BlockSpec((BR, DH), lambda i: (i, 0))
    wf = pl.BlockSpec((D, D), lambda i: (0, 0))
    wh = pl.BlockSpec((DH, D), lambda i: (0, 0))
    return pl.pallas_call(
        _layer2_body,
        grid=grid,
        in_specs=[hblk, hblk, hblk, hblk, hblk, hblk, wf, wh, wh, wh, wh,
                  pl.BlockSpec((D, D_OUT), lambda i: (0, 0)),
                  pl.BlockSpec((1, D_OUT), lambda i: (0, 0))],
        out_specs=pl.BlockSpec((BR, D_OUT), lambda i: (i, 0)),
        out_shape=jax.ShapeDtypeStruct((N_NODES, D_OUT), jnp.float32),
    )(ya, yb, ha1, hb1, ha2, hb2,
      W1[0], W1[1, :DH], W1[1, DH:], W1[2, :DH], W1[2, DH:], Wm,
      bm.reshape(1, D_OUT))


def kernel(x, edge_index, edge_weight, W0, W1, Wm, bm):
    src3 = edge_index[1].astype(jnp.int32).reshape(NS, NCHUNK, CHUNK)
    dst3 = edge_index[0].astype(jnp.int32).reshape(NS, NCHUNK, CHUNK)
    ew3 = edge_weight.reshape(NS, NCHUNK, CHUNK)
    xa, xb = x[:, :DH], x[:, DH:]

    za1, zb1 = _spmm(xa, xb, src3, dst3, ew3)
    za2, zb2 = _spmm(za1, zb1, src3, dst3, ew3)
    ya, yb = _tc_layer1(x, za1, zb1, za2, zb2, W0)

    ha1, hb1 = _spmm(ya, yb, src3, dst3, ew3)
    ha2, hb2 = _spmm(ha1, hb1, src3, dst3, ew3)
    return _tc_layer2(ya, yb, ha1, hb1, ha2, hb2, W1, Wm, bm)


# speedup vs baseline: 11.8892x; 2.9705x over previous
import functools

import jax
import jax.numpy as jnp
from jax import lax
from jax.experimental import pallas as pl
from jax.experimental.pallas import tpu as pltpu
from jax.experimental.pallas import tpu_sc as plsc

N_NODES = 10000
N_EDGES = 320000
D = 128
DH = 64
D_OUT = 64

NC = 2
NS = 16
EPT = N_EDGES // NS
CHUNK = 80
NCHUNK = EPT // CHUNK
NBUF = 5
GAHEAD = 3
RPT = 640
ZROWS = 80


def _spmm_body(xa_hbm, xb_hbm, src_hbm, dst_hbm, ew_hbm, outa_hbm, outb_hbm,
               src_b, dst_b, ew_b, rows_v, zbuf_v, acc, isem, gsem, ssem):
    cid = lax.axis_index("c")
    sid = lax.axis_index("s")

    def zrow(r, carry):
        for c in range(DH // 16):
            zbuf_v[r, pl.ds(c * 16, 16)] = jnp.zeros((16,), jnp.float32)
        return carry
    lax.fori_loop(0, ZROWS, zrow, 0)
    row0 = sid * RPT

    @pl.when(sid < NS - 1)
    def _zero_full():
        for j in range(RPT // ZROWS):
            pltpu.sync_copy(zbuf_v, acc.at[pl.ds(row0 + j * ZROWS, ZROWS)])

    @pl.when(sid == NS - 1)
    def _zero_last():
        for j in range(5):
            pltpu.sync_copy(zbuf_v, acc.at[pl.ds(row0 + j * ZROWS, ZROWS)])

    plsc.subcore_barrier()

    inv_n = jnp.float32(1.0 / N_NODES)

    def _issue_idx(ci, b):
        pltpu.async_copy(src_hbm.at[sid, ci], src_b[b], isem[b])
        pltpu.async_copy(dst_hbm.at[sid, ci], dst_b[b], isem[b])
        pltpu.async_copy(ew_hbm.at[sid, ci], ew_b[b], isem[b])

    def _wait_idx(ci, b):
        pltpu.make_async_copy(src_hbm.at[sid, ci], src_b[b], isem[b]).wait()
        pltpu.make_async_copy(dst_hbm.at[sid, ci], dst_b[b], isem[b]).wait()
        pltpu.make_async_copy(ew_hbm.at[sid, ci], ew_b[b], isem[b]).wait()

    def _issue_gather(b):
        @pl.when(cid == 0)
        def _ga():
            pltpu.async_copy(xa_hbm.at[src_b[b]], rows_v[b], gsem[b])

        @pl.when(cid == 1)
        def _gb():
            pltpu.async_copy(xb_hbm.at[src_b[b]], rows_v[b], gsem[b])

    def _wait_gather(b):
        @pl.when(cid == 0)
        def _wa():
            pltpu.make_async_copy(xa_hbm.at[src_b[b]], rows_v[b],
                                  gsem[b]).wait()

        @pl.when(cid == 1)
        def _wb():
            pltpu.make_async_copy(xb_hbm.at[src_b[b]], rows_v[b],
                                  gsem[b]).wait()

    def _wait_scatter(b):
        pltpu.make_async_copy(rows_v[b], acc.at[dst_b[b]], ssem[b]).wait()

    for j in range(NBUF):
        _issue_idx(j, j)
    for j in range(GAHEAD):
        _wait_idx(j, j)
        _issue_gather(j)

    def ring_body(cg, carry):
        for b in range(NBUF):
            ci = cg * NBUF + b
            _wait_gather(b)

            bg = (b + GAHEAD) % NBUF

            @pl.when(ci + GAHEAD < NCHUNK)
            def _launch_ahead():
                _wait_idx(ci + GAHEAD, bg)

                @pl.when(ci >= NBUF - GAHEAD)
                def _reuse_guard():
                    _wait_scatter(bg)

                _issue_gather(bg)

            def group_body(g, inner):
                ew16 = ew_b[b][pl.ds(g * 16, 16)] * inv_n
                for j0 in range(0, 16, 4):
                    es = [g * 16 + j0 + t for t in range(4)]
                    ws = [ew16[j0 + t] for t in range(4)]
                    vals = [rows_v[b][es[t], pl.ds(c * 16, 16)]
                            for t in range(4) for c in range(DH // 16)]
                    for t in range(4):
                        for c in range(DH // 16):
                            rows_v[b][es[t], pl.ds(c * 16, 16)] = (
                                vals[t * (DH // 16) + c] * ws[t])
                return inner
            lax.fori_loop(0, CHUNK // 16, group_body, 0)

            pltpu.async_copy(rows_v[b], acc.at[dst_b[b]], ssem[b], add=True)

            @pl.when(ci + NBUF < NCHUNK)
            def _prefetch_idx():
                _issue_idx(ci + NBUF, b)
        return carry
    lax.fori_loop(0, NCHUNK // NBUF, ring_body, 0)

    for b in range(NBUF):
        _wait_scatter(b)

    plsc.subcore_barrier()

    @pl.when(cid == 0)
    def _outa():
        @pl.when(sid < NS - 1)
        def _full():
            pltpu.sync_copy(acc.at[pl.ds(row0, RPT)],
                            outa_hbm.at[pl.ds(row0, RPT)])

        @pl.when(sid == NS - 1)
        def _last():
            pltpu.sync_copy(acc.at[pl.ds(row0, 400)],
                            outa_hbm.at[pl.ds(row0, 400)])

    @pl.when(cid == 1)
    def _outb():
        @pl.when(sid < NS - 1)
        def _full():
            pltpu.sync_copy(acc.at[pl.ds(row0, RPT)],
                            outb_hbm.at[pl.ds(row0, RPT)])

        @pl.when(sid == NS - 1)
        def _last():
            pltpu.sync_copy(acc.at[pl.ds(row0, 400)],
                            outb_hbm.at[pl.ds(row0, 400)])


def _spmm(xa, xb, src3, dst3, ew3):
    mesh = plsc.VectorSubcoreMesh(core_axis_name="c", subcore_axis_name="s")
    f = pl.kernel(
        _spmm_body,
        compiler_params=pltpu.CompilerParams(use_tc_tiling_on_sc=False),
        out_type=(jax.ShapeDtypeStruct((N_NODES, DH), jnp.float32),
                  jax.ShapeDtypeStruct((N_NODES, DH), jnp.float32)),
        mesh=mesh,
        scratch_types=[
            [pltpu.VMEM((CHUNK,), jnp.int32)] * NBUF,
            [pltpu.VMEM((CHUNK,), jnp.int32)] * NBUF,
            [pltpu.VMEM((CHUNK,), jnp.float32)] * NBUF,
            [pltpu.VMEM((CHUNK, DH), jnp.float32)] * NBUF,
            pltpu.VMEM((ZROWS, DH), jnp.float32),
            pltpu.VMEM_SHARED((N_NODES, DH), jnp.float32),
            [pltpu.SemaphoreType.DMA] * NBUF,
            [pltpu.SemaphoreType.DMA] * NBUF,
            [pltpu.SemaphoreType.DMA] * NBUF,
        ],
    )
    return f(xa, xb, src3, dst3, ew3)


BR = 1000


def _layer1_body(x_ref, za1_ref, zb1_ref, za2_ref, zb2_ref,
                 w00_ref, w01a_ref, w01b_ref, w02a_ref, w02b_ref,
                 ya_ref, yb_ref):
    def dot(a, b):
        return jnp.dot(a, b, preferred_element_type=jnp.float32)
    y = dot(x_ref[...], w00_ref[...])
    y += dot(za1_ref[...], w01a_ref[...]) + dot(zb1_ref[...], w01b_ref[...])
    y += dot(za2_ref[...], w02a_ref[...]) + dot(zb2_ref[...], w02b_ref[...])
    y = jnp.maximum(y, 0.0)
    ya_ref[...] = y[:, :DH]
    yb_ref[...] = y[:, DH:]


def _tc_layer1(x, za1, zb1, za2, zb2, W0):
    grid = (N_NODES // BR,)
    fblk = pl.BlockSpec((BR, D), lambda i: (i, 0))
    hblk = pl.BlockSpec((BR, DH), lambda i: (i, 0))
    wf = pl.BlockSpec((D, D), lambda i: (0, 0))
    wh = pl.BlockSpec((DH, D), lambda i: (0, 0))
    return pl.pallas_call(
        _layer1_body,
        grid=grid,
        in_specs=[fblk, hblk, hblk, hblk, hblk, wf, wh, wh, wh, wh],
        out_specs=[hblk, hblk],
        out_shape=[jax.ShapeDtypeStruct((N_NODES, DH), jnp.float32),
                   jax.ShapeDtypeStruct((N_NODES, DH), jnp.float32)],
    )(x, za1, zb1, za2, zb2,
      W0[0], W0[1, :DH], W0[1, DH:], W0[2, :DH], W0[2, DH:])


def _layer2_body(ya_ref, yb_ref, ha1_ref, hb1_ref, ha2_ref, hb2_ref,
                 w10_ref, w11a_ref, w11b_ref, w12a_ref, w12b_ref,
                 wm_ref, bm_ref, out_ref):
    def dot(a, b):
        return jnp.dot(a, b, preferred_element_type=jnp.float32)
    y1 = jnp.concatenate([ya_ref[...], yb_ref[...]], axis=1)
    y2 = dot(y1, w10_ref[...])
    y2 += dot(ha1_ref[...], w11a_ref[...]) + dot(hb1_ref[...], w11b_ref[...])
    y2 += dot(ha2_ref[...], w12a_ref[...]) + dot(hb2_ref[...], w12b_ref[...])
    y2 = jnp.maximum(y2, 0.0)
    out_ref[...] = dot(y2, wm_ref[...]) + bm_ref[...]


def _tc_layer2(ya, yb, ha1, hb1, ha2, hb2, W1, Wm, bm):
    grid = (N_NODES // BR,)
    hblk = pl.BlockSpec((BR, DH), lambda i: (i, 0))
    wf = pl.BlockSpec((D, D), lambda i: (0, 0))
    wh = pl.BlockSpec((DH, D), lambda i: (0, 0))
    return pl.pallas_call(
        _layer2_body,
        grid=grid,
        in_specs=[hblk, hblk, hblk, hblk, hblk, hblk, wf, wh, wh, wh, wh,
                  pl.BlockSpec((D, D_OUT), lambda i: (0, 0)),
                  pl.BlockSpec((1, D_OUT), lambda i: (0, 0))],
        out_specs=pl.BlockSpec((BR, D_OUT), lambda i: (i, 0)),
        out_shape=jax.ShapeDtypeStruct((N_NODES, D_OUT), jnp.float32),
    )(ya, yb, ha1, hb1, ha2, hb2,
      W1[0], W1[1, :DH], W1[1, DH:], W1[2, :DH], W1[2, DH:], Wm,
      bm.reshape(1, D_OUT))


def kernel(x, edge_index, edge_weight, W0, W1, Wm, bm):
    src3 = edge_index[1].astype(jnp.int32).reshape(NS, NCHUNK, CHUNK)
    dst3 = edge_index[0].astype(jnp.int32).reshape(NS, NCHUNK, CHUNK)
    ew3 = edge_weight.reshape(NS, NCHUNK, CHUNK)
    xa, xb = x[:, :DH], x[:, DH:]

    za1, zb1 = _spmm(xa, xb, src3, dst3, ew3)
    za2, zb2 = _spmm(za1, zb1, src3, dst3, ew3)
    ya, yb = _tc_layer1(x, za1, zb1, za2, zb2, W0)

    ha1, hb1 = _spmm(ya, yb, src3, dst3, ew3)
    ha2, hb2 = _spmm(ha1, hb1, src3, dst3, ew3)
    return _tc_layer2(ya, yb, ha1, hb1, ha2, hb2, W1, Wm, bm)
